# baseline (speedup 1.0000x reference)
import functools

import jax
import jax.numpy as jnp
from jax import lax
from jax.experimental import pallas as pl
from jax.experimental.pallas import tpu as pltpu
from jax.experimental.pallas import tpu_sc as plsc

_MARGIN = 1.0
_NC = 2
_NS = 16
_L = 16
_NW = _NC * _NS

_CHUNK = 8192
_UNROLL = 8


def _sc_partial_max(y_hat, y):
    n = y_hat.shape[0]
    per_w = n // _NW
    n_chunks = per_w // _CHUNK

    mesh = plsc.VectorSubcoreMesh(core_axis_name="c", subcore_axis_name="s")

    @functools.partial(
        pl.kernel,
        mesh=mesh,
        out_type=jax.ShapeDtypeStruct((_NW, 2, _L), jnp.float32),
        scratch_types=[
            pltpu.VMEM((_CHUNK,), jnp.float32),
            pltpu.VMEM((_CHUNK,), jnp.float32),
            pltpu.VMEM((_CHUNK,), jnp.int32),
            pltpu.VMEM((_CHUNK,), jnp.int32),
            pltpu.VMEM((_L,), jnp.float32),
            pltpu.VMEM((_L,), jnp.float32),
            pltpu.SemaphoreType.DMA,
            pltpu.SemaphoreType.DMA,
        ],
    )
    def k(yh_hbm, y_hbm, out_hbm, yh0, yh1, y0, y1, pv, nv, sem0, sem1):
        wid = lax.axis_index("c") * _NS + lax.axis_index("s")
        base = wid * per_w
        yh_bufs = (yh0, yh1)
        y_bufs = (y0, y1)
        sems = (sem0, sem1)

        neg = jnp.full((_L,), -jnp.inf, dtype=jnp.float32)

        copies = []
        for b in range(2):
            off = base + b * _CHUNK
            copies.append((
                pltpu.async_copy(yh_hbm.at[pl.ds(off, _CHUNK)], yh_bufs[b], sems[b]),
                pltpu.async_copy(y_hbm.at[pl.ds(off, _CHUNK)], y_bufs[b], sems[b]),
            ))

        def chunk_reduce(yhb, yb, pacc, nacc):
            def body(i, carry):
                pacc, nacc = carry
                for j in range(_UNROLL):
                    o = (i * _UNROLL + j) * _L
                    yh = yhb[pl.ds(o, _L)]
                    yv = yb[pl.ds(o, _L)]
                    m = yv > 0
                    pacc = jnp.maximum(pacc, jnp.where(m, yh, neg))
                    nacc = jnp.maximum(nacc, jnp.where(m, neg, yh))
                return pacc, nacc
            return lax.fori_loop(0, _CHUNK // (_L * _UNROLL), body, (pacc, nacc))

        pacc = neg
        nacc = neg
        for c in range(n_chunks):
            b = c % 2
            c_yh, c_y = copies[b]
            c_yh.wait()
            c_y.wait()
            pacc, nacc = chunk_reduce(yh_bufs[b], y_bufs[b], pacc, nacc)
            if c + 2 < n_chunks:
                off = base + (c + 2) * _CHUNK
                copies[b] = (
                    pltpu.async_copy(yh_hbm.at[pl.ds(off, _CHUNK)], yh_bufs[b], sems[b]),
                    pltpu.async_copy(y_hbm.at[pl.ds(off, _CHUNK)], y_bufs[b], sems[b]),
                )

        pv[...] = pacc
        nv[...] = nacc
        pltpu.sync_copy(pv, out_hbm.at[wid, 0])
        pltpu.sync_copy(nv, out_hbm.at[wid, 1])

    return k(y_hat, y)


def kernel(y_hat, y):
    parts = _sc_partial_max(y_hat, y.astype(jnp.int32))
    pos_max = jnp.max(parts[:, 0, :])
    neg_max = jnp.max(parts[:, 1, :])
    return jax.nn.relu(jnp.float32(_MARGIN) - pos_max + neg_max)


# speedup vs baseline: 1.0120x; 1.0120x over previous
import functools

import jax
import jax.numpy as jnp
from jax import lax
from jax.experimental import pallas as pl
from jax.experimental.pallas import tpu as pltpu
from jax.experimental.pallas import tpu_sc as plsc

_MARGIN = 1.0
_NC = 2
_NS = 16
_L = 16
_NW = _NC * _NS

_CHUNK = 8192
_UNROLL = 8


def _sc_partial_max(y_hat, y):
    n = y_hat.shape[0]
    per_w = n // _NW
    n_chunks = per_w // _CHUNK

    mesh = plsc.VectorSubcoreMesh(core_axis_name="c", subcore_axis_name="s")

    @functools.partial(
        pl.kernel,
        mesh=mesh,
        out_type=jax.ShapeDtypeStruct((_NW, 2, _L), jnp.float32),
        scratch_types=[
            pltpu.VMEM((_CHUNK,), jnp.float32),
            pltpu.VMEM((_CHUNK,), jnp.float32),
            pltpu.VMEM((_CHUNK,), jnp.int32),
            pltpu.VMEM((_CHUNK,), jnp.int32),
            pltpu.VMEM((_L,), jnp.float32),
            pltpu.VMEM((_L,), jnp.float32),
            pltpu.SemaphoreType.DMA,
            pltpu.SemaphoreType.DMA,
        ],
    )
    def k(yh_hbm, y_hbm, out_hbm, yh0, yh1, y0, y1, pv, nv, sem0, sem1):
        wid = lax.axis_index("c") * _NS + lax.axis_index("s")
        base = wid * per_w
        yh_bufs = (yh0, yh1)
        y_bufs = (y0, y1)
        sems = (sem0, sem1)

        neg = jnp.full((_L,), -jnp.inf, dtype=jnp.float32)

        copies = []
        for b in range(2):
            off = base + b * _CHUNK
            copies.append((
                pltpu.async_copy(yh_hbm.at[pl.ds(off, _CHUNK)], yh_bufs[b], sems[b]),
                pltpu.async_copy(y_hbm.at[pl.ds(off, _CHUNK)], y_bufs[b], sems[b]),
            ))

        def chunk_reduce(yhb, yb, accs):
            def body(i, accs):
                out = []
                for j in range(_UNROLL):
                    o = (i * _UNROLL + j) * _L
                    yh = yhb[pl.ds(o, _L)]
                    yv = yb[pl.ds(o, _L)]
                    m = yv > 0
                    p, q = accs[j]
                    out.append((
                        jnp.maximum(p, jnp.where(m, yh, neg)),
                        jnp.maximum(q, jnp.where(m, neg, yh)),
                    ))
                return tuple(out)
            return plsc.parallel_loop(
                0, _CHUNK // (_L * _UNROLL), 1, unroll=2, carry=accs)(body)

        accs = tuple((neg, neg) for _ in range(_UNROLL))
        for c in range(n_chunks):
            b = c % 2
            c_yh, c_y = copies[b]
            c_yh.wait()
            c_y.wait()
            accs = chunk_reduce(yh_bufs[b], y_bufs[b], accs)
            if c + 2 < n_chunks:
                off = base + (c + 2) * _CHUNK
                copies[b] = (
                    pltpu.async_copy(yh_hbm.at[pl.ds(off, _CHUNK)], yh_bufs[b], sems[b]),
                    pltpu.async_copy(y_hbm.at[pl.ds(off, _CHUNK)], y_bufs[b], sems[b]),
                )

        pacc = accs[0][0]
        nacc = accs[0][1]
        for j in range(1, _UNROLL):
            pacc = jnp.maximum(pacc, accs[j][0])
            nacc = jnp.maximum(nacc, accs[j][1])

        pv[...] = pacc
        nv[...] = nacc
        pltpu.sync_copy(pv, out_hbm.at[wid, 0])
        pltpu.sync_copy(nv, out_hbm.at[wid, 1])

    return k(y_hat, y)


def kernel(y_hat, y):
    parts = _sc_partial_max(y_hat, y.astype(jnp.int32))
    pos_max = jnp.max(parts[:, 0, :])
    neg_max = jnp.max(parts[:, 1, :])
    return jax.nn.relu(jnp.float32(_MARGIN) - pos_max + neg_max)
